# baseline (speedup 1.0000x reference)
import functools

import jax
import jax.numpy as jnp
from jax import lax
from jax.experimental import pallas as pl
from jax.experimental.pallas import tpu as pltpu

try:
    from jax.experimental.pallas import tpu_sc as plsc
    _info = plsc.get_sparse_core_info()
    _NC, _NS = _info.num_cores, _info.num_subcores
except Exception:
    plsc = None
    _NC, _NS = 2, 16

_B = 16384
_UEMB = 64
_NW = _NC * _NS
_BPW = _B // _NW


def _make_sc_gather():
    mesh = plsc.VectorSubcoreMesh(core_axis_name="c", subcore_axis_name="s")

    @functools.partial(
        pl.kernel,
        mesh=mesh,
        out_type=jax.ShapeDtypeStruct((_B, _UEMB), jnp.float32),
        scratch_types=[
            pltpu.VMEM((_BPW,), jnp.int32),
            pltpu.VMEM((_BPW, _UEMB), jnp.float32),
            pltpu.SemaphoreType.DMA,
        ],
        compiler_params=pltpu.CompilerParams(use_tc_tiling_on_sc=False),
    )
    def sc_gather(table_hbm, idx_hbm, out_hbm, idx_v, rows_v, sem):
        wid = lax.axis_index("s") * _NC + lax.axis_index("c")
        base = wid * _BPW
        pltpu.sync_copy(idx_hbm.at[pl.ds(base, _BPW)], idx_v)
        pltpu.async_copy(table_hbm.at[idx_v], rows_v, sem).wait()
        pltpu.sync_copy(rows_v, out_hbm.at[pl.ds(base, _BPW)])

    return sc_gather


def _tc_body(euid_ref, city_ref, d_ref, ts_ref, tc_ref, poi_ref,
             cityw_ref, dayw_ref, dayb_ref, timew_ref, timeb_ref,
             poiw_ref, poib_ref, out_ref):
    euid = euid_ref[...]
    cityv = city_ref[...]
    cityw = cityw_ref[...]
    e_city = jnp.zeros((euid.shape[0], 4), jnp.float32)
    for c in range(4):
        e_city = e_city + (cityv == c).astype(jnp.float32) * cityw[c:c + 1, :]
    e_day = jnp.maximum(d_ref[...] * dayw_ref[...] + dayb_ref[...], 0.0)
    e_time = jnp.maximum(
        ts_ref[...] * timew_ref[0:1, :] + tc_ref[...] * timew_ref[1:2, :]
        + timeb_ref[...], 0.0)
    e_poi = jnp.maximum(
        jnp.dot(poi_ref[...], poiw_ref[...],
                preferred_element_type=jnp.float32) + poib_ref[...], 0.0)
    out_ref[...] = jnp.concatenate([euid, e_city, e_day, e_time, e_poi],
                                   axis=1)


def _tc_dense(e_uid, city2, d2, ts2, tc2, poi_norm,
              cityw, dayw, dayb, timew, timeb, poiw, poib):
    bm = 2048
    grid = (_B // bm,)
    row = lambda i: (i, 0)
    rep = lambda i: (0, 0)
    return pl.pallas_call(
        _tc_body,
        grid=grid,
        in_specs=[
            pl.BlockSpec((bm, _UEMB), row),
            pl.BlockSpec((bm, 1), row),
            pl.BlockSpec((bm, 1), row),
            pl.BlockSpec((bm, 1), row),
            pl.BlockSpec((bm, 1), row),
            pl.BlockSpec((bm, 85), row),
            pl.BlockSpec((4, 4), rep),
            pl.BlockSpec((1, 8), rep),
            pl.BlockSpec((1, 8), rep),
            pl.BlockSpec((2, 8), rep),
            pl.BlockSpec((1, 8), rep),
            pl.BlockSpec((85, 10), rep),
            pl.BlockSpec((1, 10), rep),
        ],
        out_specs=pl.BlockSpec((bm, 94), row),
        out_shape=jax.ShapeDtypeStruct((_B, 94), jnp.float32),
    )(e_uid, city2, d2, ts2, tc2, poi_norm,
      cityw, dayw, dayb, timew, timeb, poiw, poib)


def kernel(uid, d_norm, t_sin, t_cos, city, poi_norm,
           uid_emb_W, city_emb_W, day_W, day_b, time_W, time_b,
           poi_W, poi_b):
    e_uid = _make_sc_gather()(uid_emb_W, uid.astype(jnp.int32))
    return _tc_dense(
        e_uid,
        city.astype(jnp.int32).reshape(_B, 1),
        d_norm.reshape(_B, 1),
        t_sin.reshape(_B, 1),
        t_cos.reshape(_B, 1),
        poi_norm,
        city_emb_W,
        day_W.T,
        day_b.reshape(1, 8),
        time_W.T,
        time_b.reshape(1, 8),
        poi_W.T,
        poi_b.reshape(1, 10),
    )


# speedup vs baseline: 2.3147x; 2.3147x over previous
import functools

import jax
import jax.numpy as jnp
from jax import lax
from jax.experimental import pallas as pl
from jax.experimental.pallas import tpu as pltpu

try:
    from jax.experimental.pallas import tpu_sc as plsc
    _info = plsc.get_sparse_core_info()
    _NC, _NS = _info.num_cores, _info.num_subcores
except Exception:
    plsc = None
    _NC, _NS = 2, 16

_B = 16384
_UEMB = 64
_NUSERS = 1000000
_NW = _NC * _NS
_BPW = _B // _NW


_G = 16
_NGRP = _BPW // _G
_LAG = 2


def _make_sc_gather():
    mesh = plsc.VectorSubcoreMesh(core_axis_name="c", subcore_axis_name="s")

    @functools.partial(
        pl.kernel,
        mesh=mesh,
        out_type=jax.ShapeDtypeStruct((_B, _UEMB), jnp.float32),
        scratch_types=[
            pltpu.VMEM((_BPW,), jnp.int32),
            pltpu.VMEM((_BPW, _UEMB), jnp.float32),
            pltpu.SemaphoreType.DMA,
        ],
        compiler_params=pltpu.CompilerParams(use_tc_tiling_on_sc=True,
                                             needs_layout_passes=False),
    )
    def sc_gather(table_hbm, idx_hbm, out_hbm, idx_v, rows_v, sem):
        wid = lax.axis_index("s") * _NC + lax.axis_index("c")
        base = wid * _BPW
        lanes = lax.iota(jnp.int32, 16)
        pltpu.sync_copy(idx_hbm.at[pl.ds(base, _BPW)], idx_v)

        def fire(g):
            v = idx_v[pl.ds(g * _G, _G)]
            for j in range(_G):
                u = lax.reduce_max(jnp.where(lanes == j, v, -1), (0,))
                t = lax.shift_right_logical(u, 3)
                s = jnp.bitwise_and(u, 7)
                pltpu.async_copy(table_hbm.at[t, s], rows_v.at[g * _G + j],
                                 sem)

        def drain():
            for j in range(_G):
                pltpu.make_async_copy(table_hbm.at[0, 0], rows_v.at[0],
                                      sem).wait()

        def body(g, carry):
            fire(g)

            @pl.when(g >= _LAG)
            def _():
                drain()

            return carry

        lax.fori_loop(0, _NGRP, body, 0)
        for _ in range(_LAG):
            drain()
        pltpu.sync_copy(rows_v, out_hbm.at[pl.ds(base, _BPW)])

    return sc_gather


def _tc_body(euid_ref, city_ref, d_ref, ts_ref, tc_ref, poi_ref,
             cityw_ref, dayw_ref, dayb_ref, timew_ref, timeb_ref,
             poiw_ref, poib_ref, out_ref):
    euid = euid_ref[...]
    cityv = city_ref[...]
    cityw = cityw_ref[...]
    e_city = jnp.zeros((euid.shape[0], 4), jnp.float32)
    for c in range(4):
        e_city = e_city + (cityv == c).astype(jnp.float32) * cityw[c:c + 1, :]
    e_day = jnp.maximum(d_ref[...] * dayw_ref[...] + dayb_ref[...], 0.0)
    e_time = jnp.maximum(
        ts_ref[...] * timew_ref[0:1, :] + tc_ref[...] * timew_ref[1:2, :]
        + timeb_ref[...], 0.0)
    e_poi = jnp.maximum(
        jnp.dot(poi_ref[...], poiw_ref[...],
                preferred_element_type=jnp.float32) + poib_ref[...], 0.0)
    out_ref[...] = jnp.concatenate([euid, e_city, e_day, e_time, e_poi],
                                   axis=1)


def _tc_dense(e_uid, city2, d2, ts2, tc2, poi_norm,
              cityw, dayw, dayb, timew, timeb, poiw, poib):
    bm = 2048
    grid = (_B // bm,)
    row = lambda i: (i, 0)
    rep = lambda i: (0, 0)
    return pl.pallas_call(
        _tc_body,
        grid=grid,
        in_specs=[
            pl.BlockSpec((bm, _UEMB), row),
            pl.BlockSpec((bm, 1), row),
            pl.BlockSpec((bm, 1), row),
            pl.BlockSpec((bm, 1), row),
            pl.BlockSpec((bm, 1), row),
            pl.BlockSpec((bm, 85), row),
            pl.BlockSpec((4, 4), rep),
            pl.BlockSpec((1, 8), rep),
            pl.BlockSpec((1, 8), rep),
            pl.BlockSpec((2, 8), rep),
            pl.BlockSpec((1, 8), rep),
            pl.BlockSpec((85, 10), rep),
            pl.BlockSpec((1, 10), rep),
        ],
        out_specs=pl.BlockSpec((bm, 94), row),
        out_shape=jax.ShapeDtypeStruct((_B, 94), jnp.float32),
    )(e_uid, city2, d2, ts2, tc2, poi_norm,
      cityw, dayw, dayb, timew, timeb, poiw, poib)


def kernel(uid, d_norm, t_sin, t_cos, city, poi_norm,
           uid_emb_W, city_emb_W, day_W, day_b, time_W, time_b,
           poi_W, poi_b):
    table3 = uid_emb_W.reshape(_NUSERS // 8, 8, _UEMB)
    e_uid = _make_sc_gather()(table3, uid.astype(jnp.int32))
    return _tc_dense(
        e_uid,
        city.astype(jnp.int32).reshape(_B, 1),
        d_norm.reshape(_B, 1),
        t_sin.reshape(_B, 1),
        t_cos.reshape(_B, 1),
        poi_norm,
        city_emb_W,
        day_W.T,
        day_b.reshape(1, 8),
        time_W.T,
        time_b.reshape(1, 8),
        poi_W.T,
        poi_b.reshape(1, 10),
    )
